# baseline (speedup 1.0000x reference)
import functools

import jax
import jax.numpy as jnp
from jax import lax
from jax.experimental import pallas as pl
from jax.experimental.pallas import tpu as pltpu
from jax.experimental.pallas import tpu_sc as plsc

N_NODES = 10000
N_EDGES = 320000
D_IN = 128
D_HID = 128
N_CLASSES = 40

NP = 10240
DW = 16
NF = NP * DW // 128

NCORE = 2
NSUB = 16
CH = 128
CPT = 80
EPT = CPT * CH
EP = NCORE * NSUB * EPT

BR = 512
GRID = NP // BR


@functools.lru_cache(maxsize=None)
def _make_deg():
    mesh = plsc.VectorSubcoreMesh(
        core_axis_name="c", subcore_axis_name="s",
        num_cores=NCORE, num_subcores=NSUB)
    rows_per_tile = NP // NSUB
    frows_per_tile = rows_per_tile * DW // 128

    out_type = [jax.ShapeDtypeStruct((NCORE, NF, 128), jnp.float32)]
    scratch = [
        pltpu.VMEM((CH,), jnp.int32),
        pltpu.VMEM((CH, DW), jnp.float32),
        pltpu.VMEM((DW, 128), jnp.float32),
        pltpu.VMEM_SHARED((NP, DW), jnp.float32),
    ]

    def degk(dst_hbm, degf_hbm, didx, ones, stage, dacc):
        cid = lax.axis_index("c")
        sid = lax.axis_index("s")
        row0 = sid * rows_per_tile
        zv = jnp.zeros((16,), jnp.float32)
        ov = jnp.ones((16,), jnp.float32)

        def z16(i, _):
            ones[i, pl.ds(0, 16)] = zv
            return 0
        lax.fori_loop(0, CH, z16, 0)

        def dzr(r, _):
            pltpu.sync_copy(ones.at[pl.ds(0, CH)],
                            dacc.at[pl.ds(row0 + r * CH, CH)])
            return 0
        lax.fori_loop(0, rows_per_tile // CH, dzr, 0)

        def o16(i, _):
            ones[i, pl.ds(0, 16)] = ov
            return 0
        lax.fori_loop(0, CH, o16, 0)
        plsc.subcore_barrier()

        base = (cid * NSUB + sid) * EPT

        def body(i, _):
            pltpu.sync_copy(dst_hbm.at[pl.ds(base + i * CH, CH)], didx)
            pltpu.sync_copy(ones, dacc.at[didx], add=True)
            return 0
        lax.fori_loop(0, CPT, body, 0)
        plsc.subcore_barrier()

        def dp(p, _):
            pltpu.sync_copy(dacc.at[pl.ds(row0 + p * CH, CH)], ones)

            def di(i, _):
                stage[i // 8, pl.ds((i % 8) * 16, 16)] = ones[i, pl.ds(0, 16)]
                return 0
            lax.fori_loop(0, CH, di, 0)
            pltpu.sync_copy(
                stage,
                degf_hbm.at[cid, pl.ds(sid * frows_per_tile + p * DW, DW)])
            return 0
        lax.fori_loop(0, rows_per_tile // CH, dp, 0)

    return pl.kernel(
        degk, out_type=out_type, mesh=mesh, scratch_types=scratch,
        compiler_params=pltpu.CompilerParams(use_tc_tiling_on_sc=False))


@functools.lru_cache(maxsize=None)
def _make_segsum():
    d = D_HID
    mesh = plsc.VectorSubcoreMesh(
        core_axis_name="c", subcore_axis_name="s",
        num_cores=NCORE, num_subcores=NSUB)
    rows_per_tile = NP // NSUB
    zrows = 32
    nz = rows_per_tile // zrows

    out_type = [jax.ShapeDtypeStruct((NCORE, NP, d), jnp.float32)]
    scratch = [
        pltpu.VMEM((CH,), jnp.int32),
        pltpu.VMEM((CH,), jnp.int32),
        pltpu.VMEM((CH, d), jnp.float32),
        pltpu.VMEM_SHARED((NP, d), jnp.float32),
        pltpu.SemaphoreType.DMA,
    ]

    def seg(src_hbm, dst_hbm, table_hbm, out_hbm, sidx, didx, rows, acc, sem):
        cid = lax.axis_index("c")
        sid = lax.axis_index("s")
        zslice = rows.at[pl.ds(0, zrows)]
        zv = jnp.zeros((16,), jnp.float32)
        row0 = sid * rows_per_tile

        def zi(i, _):
            def zj(j, _):
                rows[i, pl.ds(j * 16, 16)] = zv
                return 0
            return lax.fori_loop(0, d // 16, zj, 0)
        lax.fori_loop(0, zrows, zi, 0)

        def zr(r, _):
            pltpu.sync_copy(zslice, acc.at[pl.ds(row0 + r * zrows, zrows)])
            return 0
        lax.fori_loop(0, nz, zr, 0)
        plsc.subcore_barrier()

        base = (cid * NSUB + sid) * EPT

        def body(i, _):
            off = base + i * CH
            pltpu.sync_copy(src_hbm.at[pl.ds(off, CH)], sidx)
            pltpu.sync_copy(dst_hbm.at[pl.ds(off, CH)], didx)
            pltpu.async_copy(table_hbm.at[sidx], rows, sem).wait()
            pltpu.sync_copy(rows, acc.at[didx], add=True)
            return 0
        lax.fori_loop(0, CPT, body, 0)
        plsc.subcore_barrier()

        def co(r, _):
            rr = row0 + r * zrows
            pltpu.sync_copy(acc.at[pl.ds(rr, zrows)], zslice)
            pltpu.sync_copy(zslice, out_hbm.at[cid, pl.ds(rr, zrows)])
            return 0
        lax.fori_loop(0, nz, co, 0)

    return pl.kernel(seg, out_type=out_type, mesh=mesh, scratch_types=scratch)


def _rdeg_from_flat(dv):
    f = dv[0] + dv[1]
    fr = f.shape[0]
    ci = lax.broadcasted_iota(jnp.int32, (128, 8), 0)
    gi = lax.broadcasted_iota(jnp.int32, (128, 8), 1)
    g = jnp.where(ci // 16 == gi, 1.0 / 16.0, 0.0)
    d8 = jnp.dot(f, g, preferred_element_type=jnp.float32)
    ni = lax.broadcasted_iota(jnp.int32, (8 * fr, fr), 0)
    ri = lax.broadcasted_iota(jnp.int32, (8 * fr, fr), 1)
    a = jnp.where(ni // 8 == ri, 1.0, 0.0)
    dn8 = jnp.dot(a, d8, preferred_element_type=jnp.float32)
    n2 = lax.broadcasted_iota(jnp.int32, (8 * fr, 8), 0)
    c2 = lax.broadcasted_iota(jnp.int32, (8 * fr, 8), 1)
    sel = jnp.where(n2 % 8 == c2, 1.0, 0.0)
    deg = jnp.sum(dn8 * sel, axis=1, keepdims=True)
    return 1.0 / jnp.maximum(deg, 1.0)


def _mm1_body(x_ref, w_ref, y_ref):
    y_ref[...] = jnp.dot(x_ref[...], w_ref[...],
                         preferred_element_type=jnp.float32)


def _mm1(xp, w1):
    return pl.pallas_call(
        _mm1_body,
        grid=(GRID,),
        in_specs=[
            pl.BlockSpec((BR, D_IN), lambda i: (i, 0)),
            pl.BlockSpec((D_IN, D_HID), lambda i: (0, 0)),
        ],
        out_specs=pl.BlockSpec((BR, D_HID), lambda i: (i, 0)),
        out_shape=jax.ShapeDtypeStruct((NP, D_HID), jnp.float32),
    )(xp, w1)


def _fused_body(acc_ref, degf_ref, b_ref, w_ref, y_ref):
    v = acc_ref[...]
    rdeg = _rdeg_from_flat(degf_ref[...])
    h = jnp.maximum((v[0] + v[1]) * rdeg + b_ref[0:1, :], 0.0)
    y_ref[...] = jnp.dot(h, w_ref[...], preferred_element_type=jnp.float32)


def _fused(acc, degf, bb, w):
    return pl.pallas_call(
        _fused_body,
        grid=(GRID,),
        in_specs=[
            pl.BlockSpec((NCORE, BR, D_HID), lambda i: (0, i, 0)),
            pl.BlockSpec((NCORE, BR // 8, 128), lambda i: (0, i, 0)),
            pl.BlockSpec((8, D_HID), lambda i: (0, 0)),
            pl.BlockSpec((D_HID, D_HID), lambda i: (0, 0)),
        ],
        out_specs=pl.BlockSpec((BR, D_HID), lambda i: (i, 0)),
        out_shape=jax.ShapeDtypeStruct((NP, D_HID), jnp.float32),
    )(acc, degf, bb, w)


def _final_body(acc_ref, degf_ref, b_ref, y_ref):
    v = acc_ref[...]
    rdeg = _rdeg_from_flat(degf_ref[...])
    y_ref[...] = (v[0] + v[1]) * rdeg + b_ref[0:1, :]


def _final(acc3, degf, b3b):
    return pl.pallas_call(
        _final_body,
        grid=(GRID,),
        in_specs=[
            pl.BlockSpec((NCORE, BR, D_HID), lambda i: (0, i, 0)),
            pl.BlockSpec((NCORE, BR // 8, 128), lambda i: (0, i, 0)),
            pl.BlockSpec((8, D_HID), lambda i: (0, 0)),
        ],
        out_specs=pl.BlockSpec((BR, D_HID), lambda i: (i, 0)),
        out_shape=jax.ShapeDtypeStruct((NP, D_HID), jnp.float32),
    )(acc3, degf, b3b)


def kernel(graph, features, W1, b1, W2, b2, W3, b3):
    src = graph[0].astype(jnp.int32)
    dst = graph[1].astype(jnp.int32)
    pad = EP - N_EDGES
    src_p = jnp.concatenate([src, jnp.zeros((pad,), jnp.int32)])
    pad_dst = N_NODES + jnp.arange(pad, dtype=jnp.int32) % (NP - N_NODES)
    dst_p = jnp.concatenate([dst, pad_dst])

    xp = jnp.pad(features, ((0, NP - N_NODES), (0, 0)))
    w3p = jnp.pad(W3, ((0, 0), (0, D_HID - N_CLASSES)))
    b1b = jnp.broadcast_to(b1, (8, D_HID))
    b2b = jnp.broadcast_to(b2, (8, D_HID))
    b3b = jnp.broadcast_to(jnp.pad(b3, (0, D_HID - N_CLASSES)), (8, D_HID))

    degf, = _make_deg()(dst_p)
    y1 = _mm1(xp, W1)
    acc1, = _make_segsum()(src_p, dst_p, y1)
    y2 = _fused(acc1, degf, b1b, W2)
    acc2, = _make_segsum()(src_p, dst_p, y2)
    y3 = _fused(acc2, degf, b2b, w3p)
    acc3, = _make_segsum()(src_p, dst_p, y3)
    out = _final(acc3, degf, b3b)
    return out[:N_NODES, :N_CLASSES]


# speedup vs baseline: 1.6858x; 1.6858x over previous
import functools

import jax
import jax.numpy as jnp
from jax import lax
from jax.experimental import pallas as pl
from jax.experimental.pallas import tpu as pltpu
from jax.experimental.pallas import tpu_sc as plsc

N_NODES = 10000
N_EDGES = 320000
D_IN = 128
D_HID = 128
N_CLASSES = 40

NP = 10240
DW = 16
NF = NP * DW // 128

NCORE = 2
NSUB = 16
CH = 128
CPT = 79
EPT = CPT * CH
EP = NCORE * NSUB * EPT

BR = 512
GRID = NP // BR


@functools.lru_cache(maxsize=None)
def _make_deg():
    mesh = plsc.VectorSubcoreMesh(
        core_axis_name="c", subcore_axis_name="s",
        num_cores=NCORE, num_subcores=NSUB)
    rows_per_tile = NP // NSUB
    frows_per_tile = rows_per_tile * DW // 128

    out_type = [jax.ShapeDtypeStruct((NCORE, NF, 128), jnp.float32)]
    scratch = [
        pltpu.VMEM((CH,), jnp.int32),
        pltpu.VMEM((CH, DW), jnp.float32),
        pltpu.VMEM((DW, 128), jnp.float32),
        pltpu.VMEM_SHARED((NP, DW), jnp.float32),
    ]

    def degk(dst_hbm, degf_hbm, didx, ones, stage, dacc):
        cid = lax.axis_index("c")
        sid = lax.axis_index("s")
        row0 = sid * rows_per_tile
        zv = jnp.zeros((16,), jnp.float32)
        ov = jnp.ones((16,), jnp.float32)

        def z16(i, _):
            ones[i, pl.ds(0, 16)] = zv
            return 0
        lax.fori_loop(0, CH, z16, 0)

        def dzr(r, _):
            pltpu.sync_copy(ones.at[pl.ds(0, CH)],
                            dacc.at[pl.ds(row0 + r * CH, CH)])
            return 0
        lax.fori_loop(0, rows_per_tile // CH, dzr, 0)

        def o16(i, _):
            ones[i, pl.ds(0, 16)] = ov
            return 0
        lax.fori_loop(0, CH, o16, 0)
        plsc.subcore_barrier()

        base = (cid * NSUB + sid) * EPT

        def body(i, _):
            pltpu.sync_copy(dst_hbm.at[pl.ds(base + i * CH, CH)], didx)
            pltpu.sync_copy(ones, dacc.at[didx], add=True)
            return 0
        lax.fori_loop(0, CPT, body, 0)
        plsc.subcore_barrier()

        def dp(p, _):
            pltpu.sync_copy(dacc.at[pl.ds(row0 + p * CH, CH)], ones)

            def di(i, _):
                stage[i // 8, pl.ds((i % 8) * 16, 16)] = ones[i, pl.ds(0, 16)]
                return 0
            lax.fori_loop(0, CH, di, 0)
            pltpu.sync_copy(
                stage,
                degf_hbm.at[cid, pl.ds(sid * frows_per_tile + p * DW, DW)])
            return 0
        lax.fori_loop(0, rows_per_tile // CH, dp, 0)

    return pl.kernel(
        degk, out_type=out_type, mesh=mesh, scratch_types=scratch,
        compiler_params=pltpu.CompilerParams(use_tc_tiling_on_sc=False))


@functools.lru_cache(maxsize=None)
def _make_segsum():
    d = D_HID
    mesh = plsc.VectorSubcoreMesh(
        core_axis_name="c", subcore_axis_name="s",
        num_cores=NCORE, num_subcores=NSUB)
    rows_per_tile = NP // NSUB
    zrows = 32
    nz = rows_per_tile // zrows

    out_type = [jax.ShapeDtypeStruct((NCORE, NP, d), jnp.float32)]
    scratch = [
        pltpu.VMEM((CH,), jnp.int32),
        pltpu.VMEM((CH,), jnp.int32),
        pltpu.VMEM((CH, d), jnp.float32),
        pltpu.VMEM_SHARED((NP, d), jnp.float32),
        pltpu.SemaphoreType.DMA,
    ]

    def seg(src_hbm, dst_hbm, table_hbm, out_hbm, sidx, didx, rows, acc, sem):
        cid = lax.axis_index("c")
        sid = lax.axis_index("s")
        zslice = rows.at[pl.ds(0, zrows)]
        zv = jnp.zeros((16,), jnp.float32)
        row0 = sid * rows_per_tile

        def zi(i, _):
            def zj(j, _):
                rows[i, pl.ds(j * 16, 16)] = zv
                return 0
            return lax.fori_loop(0, d // 16, zj, 0)
        lax.fori_loop(0, zrows, zi, 0)

        def zr(r, _):
            pltpu.sync_copy(zslice, acc.at[pl.ds(row0 + r * zrows, zrows)])
            return 0
        lax.fori_loop(0, nz, zr, 0)
        plsc.subcore_barrier()

        base = (cid * NSUB + sid) * EPT

        def body(i, _):
            off = base + i * CH
            pltpu.sync_copy(src_hbm.at[pl.ds(off, CH)], sidx)
            pltpu.sync_copy(dst_hbm.at[pl.ds(off, CH)], didx)
            pltpu.async_copy(table_hbm.at[sidx], rows, sem).wait()
            pltpu.sync_copy(rows, acc.at[didx], add=True)
            return 0
        lax.fori_loop(0, CPT, body, 0)
        plsc.subcore_barrier()

        def co(r, _):
            rr = row0 + r * zrows
            pltpu.sync_copy(acc.at[pl.ds(rr, zrows)], zslice)
            pltpu.sync_copy(zslice, out_hbm.at[cid, pl.ds(rr, zrows)])
            return 0
        lax.fori_loop(0, nz, co, 0)

    return pl.kernel(seg, out_type=out_type, mesh=mesh, scratch_types=scratch)


def _rdeg_from_flat(dv):
    f = dv[0] + dv[1]
    fr = f.shape[0]
    ci = lax.broadcasted_iota(jnp.int32, (128, 8), 0)
    gi = lax.broadcasted_iota(jnp.int32, (128, 8), 1)
    g = jnp.where(ci // 16 == gi, 1.0 / 16.0, 0.0)
    d8 = jnp.dot(f, g, preferred_element_type=jnp.float32)
    ni = lax.broadcasted_iota(jnp.int32, (8 * fr, fr), 0)
    ri = lax.broadcasted_iota(jnp.int32, (8 * fr, fr), 1)
    a = jnp.where(ni // 8 == ri, 1.0, 0.0)
    dn8 = jnp.dot(a, d8, preferred_element_type=jnp.float32)
    n2 = lax.broadcasted_iota(jnp.int32, (8 * fr, 8), 0)
    c2 = lax.broadcasted_iota(jnp.int32, (8 * fr, 8), 1)
    sel = jnp.where(n2 % 8 == c2, 1.0, 0.0)
    deg = jnp.sum(dn8 * sel, axis=1, keepdims=True)
    return 1.0 / jnp.maximum(deg, 1.0)


def _mm1_body(x_ref, w_ref, y_ref):
    y_ref[...] = jnp.dot(x_ref[...], w_ref[...],
                         preferred_element_type=jnp.float32)


def _mm1(xp, w1):
    return pl.pallas_call(
        _mm1_body,
        grid=(GRID,),
        in_specs=[
            pl.BlockSpec((BR, D_IN), lambda i: (i, 0)),
            pl.BlockSpec((D_IN, D_HID), lambda i: (0, 0)),
        ],
        out_specs=pl.BlockSpec((BR, D_HID), lambda i: (i, 0)),
        out_shape=jax.ShapeDtypeStruct((NP, D_HID), jnp.float32),
    )(xp, w1)


def _fused_body(acc_ref, degf_ref, b_ref, w_ref, y_ref):
    v = acc_ref[...]
    rdeg = _rdeg_from_flat(degf_ref[...])
    h = jnp.maximum((v[0] + v[1]) * rdeg + b_ref[0:1, :], 0.0)
    y_ref[...] = jnp.dot(h, w_ref[...], preferred_element_type=jnp.float32)


def _fused(acc, degf, bb, w):
    return pl.pallas_call(
        _fused_body,
        grid=(GRID,),
        in_specs=[
            pl.BlockSpec((NCORE, BR, D_HID), lambda i: (0, i, 0)),
            pl.BlockSpec((NCORE, BR // 8, 128), lambda i: (0, i, 0)),
            pl.BlockSpec((8, D_HID), lambda i: (0, 0)),
            pl.BlockSpec((D_HID, D_HID), lambda i: (0, 0)),
        ],
        out_specs=pl.BlockSpec((BR, D_HID), lambda i: (i, 0)),
        out_shape=jax.ShapeDtypeStruct((NP, D_HID), jnp.float32),
    )(acc, degf, bb, w)


def _final_body(acc_ref, degf_ref, b_ref, y_ref):
    v = acc_ref[...]
    rdeg = _rdeg_from_flat(degf_ref[...])
    y_ref[...] = (v[0] + v[1]) * rdeg + b_ref[0:1, :]


def _final(acc3, degf, b3b):
    return pl.pallas_call(
        _final_body,
        grid=(GRID,),
        in_specs=[
            pl.BlockSpec((NCORE, BR, D_HID), lambda i: (0, i, 0)),
            pl.BlockSpec((NCORE, BR // 8, 128), lambda i: (0, i, 0)),
            pl.BlockSpec((8, D_HID), lambda i: (0, 0)),
        ],
        out_specs=pl.BlockSpec((BR, D_HID), lambda i: (i, 0)),
        out_shape=jax.ShapeDtypeStruct((NP, D_HID), jnp.float32),
    )(acc3, degf, b3b)


def kernel(graph, features, W1, b1, W2, b2, W3, b3):
    src = graph[0].astype(jnp.int32)
    dst = graph[1].astype(jnp.int32)
    pad = EP - N_EDGES
    src_p = jnp.concatenate([src, jnp.zeros((pad,), jnp.int32)])
    dst_p = jnp.concatenate([dst, jnp.full((pad,), N_NODES, jnp.int32)])

    xp = jnp.pad(features, ((0, NP - N_NODES), (0, 0)))
    w3p = jnp.pad(W3, ((0, 0), (0, D_HID - N_CLASSES)))
    b1b = jnp.broadcast_to(b1, (8, D_HID))
    b2b = jnp.broadcast_to(b2, (8, D_HID))
    b3b = jnp.broadcast_to(jnp.pad(b3, (0, D_HID - N_CLASSES)), (8, D_HID))

    degf, = _make_deg()(dst_p)
    y1 = _mm1(xp, W1)
    acc1, = _make_segsum()(src_p, dst_p, y1)
    y2 = _fused(acc1, degf, b1b, W2)
    acc2, = _make_segsum()(src_p, dst_p, y2)
    y3 = _fused(acc2, degf, b2b, w3p)
    acc3, = _make_segsum()(src_p, dst_p, y3)
    out = _final(acc3, degf, b3b)
    return out[:N_NODES, :N_CLASSES]


# speedup vs baseline: 2.1794x; 1.2928x over previous
import functools

import jax
import jax.numpy as jnp
from jax import lax
from jax.experimental import pallas as pl
from jax.experimental.pallas import tpu as pltpu
from jax.experimental.pallas import tpu_sc as plsc

N_NODES = 10000
N_EDGES = 320000
D_IN = 128
D_HID = 128
N_CLASSES = 40

NP = 10240
DW = 16
NF = NP * DW // 128

NCORE = 2
NSUB = 16
CH = 128
CPT = 79
EPT = CPT * CH
EP = NCORE * NSUB * EPT

BR = 512
GRID = NP // BR


@functools.lru_cache(maxsize=None)
def _make_deg():
    mesh = plsc.VectorSubcoreMesh(
        core_axis_name="c", subcore_axis_name="s",
        num_cores=NCORE, num_subcores=NSUB)
    rows_per_tile = NP // NSUB
    frows_per_tile = rows_per_tile * DW // 128

    out_type = [jax.ShapeDtypeStruct((NCORE, NF, 128), jnp.float32)]
    scratch = [
        pltpu.VMEM((CH,), jnp.int32),
        pltpu.VMEM((CH, DW), jnp.float32),
        pltpu.VMEM((DW, 128), jnp.float32),
        pltpu.VMEM_SHARED((NP, DW), jnp.float32),
    ]

    def degk(dst_hbm, degf_hbm, didx, ones, stage, dacc):
        cid = lax.axis_index("c")
        sid = lax.axis_index("s")
        row0 = sid * rows_per_tile
        zv = jnp.zeros((16,), jnp.float32)
        ov = jnp.ones((16,), jnp.float32)

        def z16(i, _):
            ones[i, pl.ds(0, 16)] = zv
            return 0
        lax.fori_loop(0, CH, z16, 0)

        def dzr(r, _):
            pltpu.sync_copy(ones.at[pl.ds(0, CH)],
                            dacc.at[pl.ds(row0 + r * CH, CH)])
            return 0
        lax.fori_loop(0, rows_per_tile // CH, dzr, 0)

        def o16(i, _):
            ones[i, pl.ds(0, 16)] = ov
            return 0
        lax.fori_loop(0, CH, o16, 0)
        plsc.subcore_barrier()

        base = (cid * NSUB + sid) * EPT

        def body(i, _):
            pltpu.sync_copy(dst_hbm.at[pl.ds(base + i * CH, CH)], didx)
            pltpu.sync_copy(ones, dacc.at[didx], add=True)
            return 0
        lax.fori_loop(0, CPT, body, 0)
        plsc.subcore_barrier()

        def dp(p, _):
            pltpu.sync_copy(dacc.at[pl.ds(row0 + p * CH, CH)], ones)

            def di(i, _):
                stage[i // 8, pl.ds((i % 8) * 16, 16)] = ones[i, pl.ds(0, 16)]
                return 0
            lax.fori_loop(0, CH, di, 0)
            pltpu.sync_copy(
                stage,
                degf_hbm.at[cid, pl.ds(sid * frows_per_tile + p * DW, DW)])
            return 0
        lax.fori_loop(0, rows_per_tile // CH, dp, 0)

    return pl.kernel(
        degk, out_type=out_type, mesh=mesh, scratch_types=scratch,
        compiler_params=pltpu.CompilerParams(use_tc_tiling_on_sc=False))


@functools.lru_cache(maxsize=None)
def _make_segsum():
    d = D_HID
    mesh = plsc.VectorSubcoreMesh(
        core_axis_name="c", subcore_axis_name="s",
        num_cores=NCORE, num_subcores=NSUB)
    rows_per_tile = NP // NSUB
    zrows = 32
    nz = rows_per_tile // zrows

    out_type = [jax.ShapeDtypeStruct((NCORE, NP, d), jnp.float32)]
    scratch = [
        pltpu.VMEM((CH,), jnp.int32),
        pltpu.VMEM((CH,), jnp.int32),
        pltpu.VMEM((CH,), jnp.int32),
        pltpu.VMEM((CH,), jnp.int32),
        pltpu.VMEM((CH, d), jnp.float32),
        pltpu.VMEM((CH, d), jnp.float32),
        pltpu.VMEM_SHARED((NP, d), jnp.float32),
        pltpu.SemaphoreType.DMA,
        pltpu.SemaphoreType.DMA,
    ]

    def seg(src_hbm, dst_hbm, table_hbm, out_hbm,
            sidx0, sidx1, didx0, didx1, rows0, rows1, acc, gsem0, gsem1):
        sidx = (sidx0, sidx1)
        didx = (didx0, didx1)
        rows = (rows0, rows1)
        gsem = (gsem0, gsem1)
        cid = lax.axis_index("c")
        sid = lax.axis_index("s")
        zslice = rows0.at[pl.ds(0, zrows)]
        zv = jnp.zeros((16,), jnp.float32)
        row0 = sid * rows_per_tile

        def zi(i, _):
            def zj(j, _):
                rows0[i, pl.ds(j * 16, 16)] = zv
                return 0
            return lax.fori_loop(0, d // 16, zj, 0)
        lax.fori_loop(0, zrows, zi, 0)

        def zr(r, _):
            pltpu.sync_copy(zslice, acc.at[pl.ds(row0 + r * zrows, zrows)])
            return 0
        lax.fori_loop(0, nz, zr, 0)

        base = (cid * NSUB + sid) * EPT
        pltpu.sync_copy(src_hbm.at[pl.ds(base, CH)], sidx0)
        pltpu.sync_copy(dst_hbm.at[pl.ds(base, CH)], didx0)
        pltpu.async_copy(table_hbm.at[sidx0], rows0, gsem0)
        plsc.subcore_barrier()

        def it_body(it, _):
            for b in range(2):
                i = 2 * it + b
                nxt = 1 - b

                @pl.when(i < CPT)
                def _proc():
                    @pl.when(i + 1 < CPT)
                    def _prefetch():
                        off = base + (i + 1) * CH
                        pltpu.sync_copy(src_hbm.at[pl.ds(off, CH)], sidx[nxt])
                        pltpu.sync_copy(dst_hbm.at[pl.ds(off, CH)], didx[nxt])
                        pltpu.async_copy(table_hbm.at[sidx[nxt]], rows[nxt],
                                         gsem[nxt])

                    pltpu.make_async_copy(table_hbm.at[sidx[b]], rows[b],
                                          gsem[b]).wait()
                    pltpu.sync_copy(rows[b], acc.at[didx[b]], add=True)
            return 0
        lax.fori_loop(0, (CPT + 1) // 2, it_body, 0)
        plsc.subcore_barrier()

        def co(r, _):
            rr = row0 + r * zrows
            pltpu.sync_copy(acc.at[pl.ds(rr, zrows)], zslice)
            pltpu.sync_copy(zslice, out_hbm.at[cid, pl.ds(rr, zrows)])
            return 0
        lax.fori_loop(0, nz, co, 0)

    return pl.kernel(seg, out_type=out_type, mesh=mesh, scratch_types=scratch)


def _rdeg_from_flat(dv):
    f = dv[0] + dv[1]
    fr = f.shape[0]
    ci = lax.broadcasted_iota(jnp.int32, (128, 8), 0)
    gi = lax.broadcasted_iota(jnp.int32, (128, 8), 1)
    g = jnp.where(ci // 16 == gi, 1.0 / 16.0, 0.0)
    d8 = jnp.dot(f, g, preferred_element_type=jnp.float32)
    ni = lax.broadcasted_iota(jnp.int32, (8 * fr, fr), 0)
    ri = lax.broadcasted_iota(jnp.int32, (8 * fr, fr), 1)
    a = jnp.where(ni // 8 == ri, 1.0, 0.0)
    dn8 = jnp.dot(a, d8, preferred_element_type=jnp.float32)
    n2 = lax.broadcasted_iota(jnp.int32, (8 * fr, 8), 0)
    c2 = lax.broadcasted_iota(jnp.int32, (8 * fr, 8), 1)
    sel = jnp.where(n2 % 8 == c2, 1.0, 0.0)
    deg = jnp.sum(dn8 * sel, axis=1, keepdims=True)
    return 1.0 / jnp.maximum(deg, 1.0)


def _mm1_body(x_ref, w_ref, y_ref):
    y_ref[...] = jnp.dot(x_ref[...], w_ref[...],
                         preferred_element_type=jnp.float32)


def _mm1(xp, w1):
    return pl.pallas_call(
        _mm1_body,
        grid=(GRID,),
        in_specs=[
            pl.BlockSpec((BR, D_IN), lambda i: (i, 0)),
            pl.BlockSpec((D_IN, D_HID), lambda i: (0, 0)),
        ],
        out_specs=pl.BlockSpec((BR, D_HID), lambda i: (i, 0)),
        out_shape=jax.ShapeDtypeStruct((NP, D_HID), jnp.float32),
    )(xp, w1)


def _fused_body(acc_ref, degf_ref, b_ref, w_ref, y_ref):
    v = acc_ref[...]
    rdeg = _rdeg_from_flat(degf_ref[...])
    h = jnp.maximum((v[0] + v[1]) * rdeg + b_ref[0:1, :], 0.0)
    y_ref[...] = jnp.dot(h, w_ref[...], preferred_element_type=jnp.float32)


def _fused(acc, degf, bb, w):
    return pl.pallas_call(
        _fused_body,
        grid=(GRID,),
        in_specs=[
            pl.BlockSpec((NCORE, BR, D_HID), lambda i: (0, i, 0)),
            pl.BlockSpec((NCORE, BR // 8, 128), lambda i: (0, i, 0)),
            pl.BlockSpec((8, D_HID), lambda i: (0, 0)),
            pl.BlockSpec((D_HID, D_HID), lambda i: (0, 0)),
        ],
        out_specs=pl.BlockSpec((BR, D_HID), lambda i: (i, 0)),
        out_shape=jax.ShapeDtypeStruct((NP, D_HID), jnp.float32),
    )(acc, degf, bb, w)


def _final_body(acc_ref, degf_ref, b_ref, y_ref):
    v = acc_ref[...]
    rdeg = _rdeg_from_flat(degf_ref[...])
    y_ref[...] = (v[0] + v[1]) * rdeg + b_ref[0:1, :]


def _final(acc3, degf, b3b):
    return pl.pallas_call(
        _final_body,
        grid=(GRID,),
        in_specs=[
            pl.BlockSpec((NCORE, BR, D_HID), lambda i: (0, i, 0)),
            pl.BlockSpec((NCORE, BR // 8, 128), lambda i: (0, i, 0)),
            pl.BlockSpec((8, D_HID), lambda i: (0, 0)),
        ],
        out_specs=pl.BlockSpec((BR, D_HID), lambda i: (i, 0)),
        out_shape=jax.ShapeDtypeStruct((NP, D_HID), jnp.float32),
    )(acc3, degf, b3b)


def kernel(graph, features, W1, b1, W2, b2, W3, b3):
    src = graph[0].astype(jnp.int32)
    dst = graph[1].astype(jnp.int32)
    pad = EP - N_EDGES
    src_p = jnp.concatenate([src, jnp.zeros((pad,), jnp.int32)])
    dst_p = jnp.concatenate([dst, jnp.full((pad,), N_NODES, jnp.int32)])

    xp = jnp.pad(features, ((0, NP - N_NODES), (0, 0)))
    w3p = jnp.pad(W3, ((0, 0), (0, D_HID - N_CLASSES)))
    b1b = jnp.broadcast_to(b1, (8, D_HID))
    b2b = jnp.broadcast_to(b2, (8, D_HID))
    b3b = jnp.broadcast_to(jnp.pad(b3, (0, D_HID - N_CLASSES)), (8, D_HID))

    degf, = _make_deg()(dst_p)
    y1 = _mm1(xp, W1)
    acc1, = _make_segsum()(src_p, dst_p, y1)
    y2 = _fused(acc1, degf, b1b, W2)
    acc2, = _make_segsum()(src_p, dst_p, y2)
    y3 = _fused(acc2, degf, b2b, w3p)
    acc3, = _make_segsum()(src_p, dst_p, y3)
    out = _final(acc3, degf, b3b)
    return out[:N_NODES, :N_CLASSES]
